# baseline (speedup 1.0000x reference)
import functools

import jax
import jax.numpy as jnp
from jax.experimental import pallas as pl
from jax.experimental.pallas import tpu as pltpu
from jax.experimental.pallas import tpu_sc as plsc


def _sc_gather(table, ids, window=128):
    n = ids.shape[0]
    d = table.shape[1]
    ids2 = ids.reshape(1, n)
    mesh = plsc.VectorSubcoreMesh(core_axis_name="c", subcore_axis_name="s")

    @functools.partial(
        pl.kernel,
        out_type=jax.ShapeDtypeStruct((n, d), table.dtype),
        mesh=mesh,
    )
    def gather_kernel(table_hbm, idx_hbm, out_hbm):
        def body(i_vmem, o_vmem):
            pltpu.sync_copy(table_hbm.at[i_vmem.at[0]], o_vmem)

        pltpu.emit_pipeline(
            body,
            grid=(n // window,),
            in_specs=[pl.BlockSpec((1, window), index_map=lambda i: (0, i))],
            out_specs=[pl.BlockSpec((window, d), index_map=lambda i: (i, 0))],
            core_axis_name=("c", "s"),
            dimension_semantics=(pltpu.PARALLEL,),
        )(idx_hbm, out_hbm)

    return gather_kernel(table, ids2)


def _mlp_body(x_ref, w1_ref, w2_ref, o_ref):
    x = x_ref[...].astype(jnp.bfloat16)
    h = jnp.dot(x, w1_ref[...], preferred_element_type=jnp.float32)
    hb = h.astype(jnp.bfloat16)
    s = hb * jax.nn.sigmoid(hb)
    o_ref[...] = jnp.dot(s, w2_ref[...], preferred_element_type=jnp.float32)


def _mlp(x, w1t, w2t, block_m=256):
    n, d_in = x.shape
    d_out = w2t.shape[1]
    return pl.pallas_call(
        _mlp_body,
        grid=(n // block_m,),
        in_specs=[
            pl.BlockSpec((block_m, d_in), lambda i: (i, 0)),
            pl.BlockSpec(w1t.shape, lambda i: (0, 0)),
            pl.BlockSpec(w2t.shape, lambda i: (0, 0)),
        ],
        out_specs=pl.BlockSpec((block_m, d_out), lambda i: (i, 0)),
        out_shape=jax.ShapeDtypeStruct((n, d_out), jnp.float32),
        compiler_params=pltpu.CompilerParams(
            dimension_semantics=("parallel",)
        ),
    )(x, w1t, w2t)


def kernel(query_ids, entity_ids, query_table, ent_table, W1, W2):
    w1t = W1.T.astype(jnp.bfloat16)
    w2t = W2.T.astype(jnp.bfloat16)
    q_rows = _sc_gather(query_table, query_ids)
    e_rows = _sc_gather(ent_table, entity_ids)
    q_out = _mlp(q_rows, w1t, w2t)
    e_out = _mlp(e_rows, w1t, w2t)
    return (q_out, e_out)


# speedup vs baseline: 1.0704x; 1.0704x over previous
import functools

import jax
import jax.numpy as jnp
from jax.experimental import pallas as pl
from jax.experimental.pallas import tpu as pltpu
from jax.experimental.pallas import tpu_sc as plsc


def _sc_gather(table, ids, window=128):
    n = ids.shape[0]
    d = table.shape[1]
    ids2 = ids.reshape(1, n)
    mesh = plsc.VectorSubcoreMesh(core_axis_name="c", subcore_axis_name="s")

    @functools.partial(
        pl.kernel,
        out_type=jax.ShapeDtypeStruct((n, d), table.dtype),
        mesh=mesh,
    )
    def gather_kernel(table_hbm, idx_hbm, out_hbm):
        def body(i_vmem, o_vmem):
            pltpu.sync_copy(table_hbm.at[i_vmem.at[0]], o_vmem)

        pltpu.emit_pipeline(
            body,
            grid=(n // window,),
            in_specs=[pl.BlockSpec((1, window), index_map=lambda i: (0, i))],
            out_specs=[pl.BlockSpec((window, d), index_map=lambda i: (i, 0))],
            core_axis_name=("c", "s"),
            dimension_semantics=(pltpu.PARALLEL,),
        )(idx_hbm, out_hbm)

    return gather_kernel(table, ids2)


def _mlp_body(x_ref, w1_ref, w2_ref, o_ref):
    x = x_ref[...].astype(jnp.bfloat16)
    h = jnp.dot(x, w1_ref[...], preferred_element_type=jnp.float32)
    hb = h.astype(jnp.bfloat16)
    s = hb * jax.nn.sigmoid(hb)
    o_ref[...] = jnp.dot(s, w2_ref[...], preferred_element_type=jnp.float32)


def _mlp(x, w1t, w2t, block_m=1024):
    n, d_in = x.shape
    d_out = w2t.shape[1]
    return pl.pallas_call(
        _mlp_body,
        grid=(n // block_m,),
        in_specs=[
            pl.BlockSpec((block_m, d_in), lambda i: (i, 0)),
            pl.BlockSpec(w1t.shape, lambda i: (0, 0)),
            pl.BlockSpec(w2t.shape, lambda i: (0, 0)),
        ],
        out_specs=pl.BlockSpec((block_m, d_out), lambda i: (i, 0)),
        out_shape=jax.ShapeDtypeStruct((n, d_out), jnp.float32),
        compiler_params=pltpu.CompilerParams(
            dimension_semantics=("parallel",)
        ),
    )(x, w1t, w2t)


def kernel(query_ids, entity_ids, query_table, ent_table, W1, W2):
    w1t = W1.T.astype(jnp.bfloat16)
    w2t = W2.T.astype(jnp.bfloat16)
    q_rows = _sc_gather(query_table, query_ids)
    e_rows = _sc_gather(ent_table, entity_ids)
    q_out = _mlp(q_rows, w1t, w2t)
    e_out = _mlp(e_rows, w1t, w2t)
    return (q_out, e_out)
